# baseline (speedup 1.0000x reference)
import functools

import jax
import jax.numpy as jnp
from jax import lax
from jax.experimental import pallas as pl
from jax.experimental.pallas import tpu as pltpu
from jax.experimental.pallas import tpu_sc as plsc

_E = 8
_D = 1024
_N = 4096
_K = 2
_NH = _N // 2
_AH = _NH * _K
_TG = 256
_GH = _AH // _TG + _E
_SH = _GH * _TG
_CT = 512
_NCK = _NH // _CT
_TT = 512

_NW = 32
_RPW = _AH // _NW
_CH = 32
_NCH = _RPW // _CH


def _router_body(x_ref, rk_ref, rb_ref, dest_ref, gates_ref, tmeta_ref):
    x = x_ref[...]
    logits = jnp.dot(x, rk_ref[...], preferred_element_type=jnp.float32)
    logits = logits + rb_ref[...]
    m = jnp.max(logits, axis=-1, keepdims=True)
    ex = jnp.exp(logits - m)
    probs = ex / jnp.sum(ex, axis=-1, keepdims=True)

    lane = lax.broadcasted_iota(jnp.int32, probs.shape, 1)
    p1 = jnp.max(probs, axis=-1, keepdims=True)
    i1 = jnp.min(jnp.where(probs == p1, lane, _E), axis=-1, keepdims=True)
    probs2 = jnp.where(lane == i1, -jnp.inf, probs)
    p2 = jnp.max(probs2, axis=-1, keepdims=True)
    i2 = jnp.min(jnp.where(probs2 == p2, lane, _E), axis=-1, keepdims=True)
    denom = p1 + p2
    gates_ref[...] = jnp.concatenate([p1 / denom, p2 / denom], axis=1)

    oh0 = (lane == i1).astype(jnp.float32)
    oh1 = (lane == i2).astype(jnp.float32)

    r = lax.broadcasted_iota(jnp.int32, (_CT, _CT), 0)
    c = lax.broadcasted_iota(jnp.int32, (_CT, _CT), 1)
    tril = (c < r).astype(jnp.bfloat16)
    oh01 = jnp.concatenate([oh0, oh1], axis=1)
    chunks = [oh01[i * _CT:(i + 1) * _CT] for i in range(_NCK)]
    big = jnp.concatenate(chunks, axis=1).astype(jnp.bfloat16)
    scan = jnp.dot(tril, big, preferred_element_type=jnp.float32)
    carry = jnp.zeros((1, 2 * _E), jnp.float32)
    parts = []
    for i in range(_NCK):
        parts.append(scan[:, i * 2 * _E:(i + 1) * 2 * _E] + carry)
        carry = carry + jnp.sum(chunks[i], axis=0, keepdims=True)
    ranks01 = jnp.concatenate(parts, axis=0)
    cnt0 = carry[:, :_E]
    ranks0 = ranks01[:, :_E]
    ranks1 = ranks01[:, _E:] + cnt0
    counts = cnt0 + carry[:, _E:]

    ptiles = jnp.floor((counts + (_TG - 1.0)) * (1.0 / _TG))
    rr = lax.broadcasted_iota(jnp.int32, (_E, _E), 0)
    cc = lax.broadcasted_iota(jnp.int32, (_E, _E), 1)
    triu = (rr < cc).astype(jnp.bfloat16)
    tstart = jnp.dot(ptiles.astype(jnp.bfloat16), triu,
                     preferred_element_type=jnp.float32)
    toff = tstart * float(_TG)

    d0 = jnp.sum(oh0 * (toff + ranks0), axis=1, keepdims=True)
    d1 = jnp.sum(oh1 * (toff + ranks1), axis=1, keepdims=True)
    dest_ref[...] = jnp.concatenate([d0, d1], axis=1).astype(jnp.int32)

    gi = lax.broadcasted_iota(jnp.int32, (64, _E), 0).astype(jnp.float32)
    te = jnp.sum((gi >= tstart).astype(jnp.float32), axis=1, keepdims=True)
    tmeta_ref[...] = (te - 1.0).astype(jnp.int32)


def _route(xbf, rkbf, rb):
    return pl.pallas_call(
        _router_body,
        grid=(2,),
        in_specs=[
            pl.BlockSpec((_NH, _D), lambda h: (h, 0)),
            pl.BlockSpec((_D, _E), lambda h: (0, 0)),
            pl.BlockSpec((1, _E), lambda h: (0, 0)),
        ],
        out_specs=[
            pl.BlockSpec((_NH, _K), lambda h: (h, 0)),
            pl.BlockSpec((_NH, _K), lambda h: (h, 0)),
            pl.BlockSpec((64, 1), lambda h: (h, 0)),
        ],
        out_shape=[
            jax.ShapeDtypeStruct((_N, _K), jnp.int32),
            jax.ShapeDtypeStruct((_N, _K), jnp.float32),
            jax.ShapeDtypeStruct((128, 1), jnp.int32),
        ],
        compiler_params=pltpu.CompilerParams(
            dimension_semantics=("arbitrary",),
        ),
    )(xbf, rkbf, rb)


@functools.lru_cache(maxsize=None)
def _sc_kernels(half):
    mesh = plsc.VectorSubcoreMesh(core_axis_name="c", subcore_axis_name="s")
    cp = pltpu.CompilerParams(use_tc_tiling_on_sc=True)
    row0 = half * _NH

    @functools.partial(
        pl.kernel, mesh=mesh, compiler_params=cp,
        out_type=jax.ShapeDtypeStruct((_SH, _D), jnp.float32),
        scratch_types=[
            pltpu.VMEM((_NCH, _CH), jnp.int32),
            pltpu.VMEM((_CH, _D), jnp.float32),
            pltpu.VMEM((_CH, _D), jnp.float32),
            pltpu.SemaphoreType.DMA,
            pltpu.SemaphoreType.DMA,
            pltpu.SemaphoreType.DMA,
        ],
    )
    def sc_scatter(x_hbm, idx_hbm, xs_hbm, idx_v, buf_a, buf_b,
                   sem_a, sem_b, sem_w):
        wid = lax.axis_index("s") * 2 + lax.axis_index("c")
        base = wid * _RPW
        pltpu.sync_copy(idx_hbm.at[wid], idx_v)
        bufs, sems = (buf_a, buf_b), (sem_a, sem_b)

        def src(j):
            return row0 + (base + j * _CH) % _NH

        rd = [None, None]
        rd[0] = pltpu.async_copy(x_hbm.at[pl.ds(src(0), _CH)], buf_a, sem_a)
        wr = None
        for j in range(_NCH):
            rd[j % 2].wait()
            if wr is not None:
                wr.wait()
            if j + 1 < _NCH:
                rd[(j + 1) % 2] = pltpu.async_copy(
                    x_hbm.at[pl.ds(src(j + 1), _CH)],
                    bufs[(j + 1) % 2], sems[(j + 1) % 2])
            wr = pltpu.async_copy(bufs[j % 2], xs_hbm.at[idx_v.at[j]], sem_w)
        wr.wait()

    @functools.partial(
        pl.kernel, mesh=mesh, compiler_params=cp,
        out_type=jax.ShapeDtypeStruct((_AH, _D), jnp.float32),
        scratch_types=[
            pltpu.VMEM((_NCH, _CH), jnp.int32),
            pltpu.VMEM((_CH, _D), jnp.float32),
            pltpu.VMEM((_CH, _D), jnp.float32),
            pltpu.SemaphoreType.DMA,
            pltpu.SemaphoreType.DMA,
            pltpu.SemaphoreType.DMA,
        ],
    )
    def sc_gather(ys_hbm, idx_hbm, rows_hbm, idx_v, buf_a, buf_b,
                  sem_a, sem_b, sem_w):
        wid = lax.axis_index("s") * 2 + lax.axis_index("c")
        base = wid * _RPW
        pltpu.sync_copy(idx_hbm.at[wid], idx_v)
        bufs, sems = (buf_a, buf_b), (sem_a, sem_b)
        rd = [None, None]
        rd[0] = pltpu.async_copy(ys_hbm.at[idx_v.at[0]], buf_a, sem_a)
        wr = None
        for j in range(_NCH):
            rd[j % 2].wait()
            if wr is not None:
                wr.wait()
            if j + 1 < _NCH:
                rd[(j + 1) % 2] = pltpu.async_copy(
                    ys_hbm.at[idx_v.at[j + 1]],
                    bufs[(j + 1) % 2], sems[(j + 1) % 2])
            wr = pltpu.async_copy(bufs[j % 2],
                                  rows_hbm.at[pl.ds(base + j * _CH, _CH)],
                                  sem_w)
        wr.wait()

    return sc_scatter, sc_gather


def _expert_body(te_ref, xs_ref, ek_ref, eb_ref, ys_ref):
    h = jnp.dot(xs_ref[...].astype(jnp.bfloat16), ek_ref[0],
                preferred_element_type=jnp.float32)
    h = h + eb_ref[0]
    ys_ref[...] = jax.nn.gelu(h)


def _expert_gemm(tile_expert, xs, ekbf, eb):
    grid_spec = pltpu.PrefetchScalarGridSpec(
        num_scalar_prefetch=1,
        grid=(_GH,),
        in_specs=[
            pl.BlockSpec((_TG, _D), lambda g, te: (g, 0)),
            pl.BlockSpec((1, _D, _D), lambda g, te: (te[g], 0, 0)),
            pl.BlockSpec((1, 1, _D), lambda g, te: (te[g], 0, 0)),
        ],
        out_specs=pl.BlockSpec((_TG, _D), lambda g, te: (g, 0)),
    )
    return pl.pallas_call(
        _expert_body,
        grid_spec=grid_spec,
        out_shape=jax.ShapeDtypeStruct((_SH, _D), jnp.float32),
        compiler_params=pltpu.CompilerParams(
            dimension_semantics=("arbitrary",),
        ),
    )(tile_expert, xs, ekbf, eb)


def _combine_body(r0_ref, r1_ref, gates_ref, wo_ref, ob_ref, out_ref):
    g = gates_ref[...]
    comb = r0_ref[...] * g[:, 0:1] + r1_ref[...] * g[:, 1:2]
    out = jnp.dot(comb.astype(jnp.bfloat16), wo_ref[...],
                  preferred_element_type=jnp.float32)
    out_ref[...] = out + ob_ref[...]


def _combine(rows, gates, wobf, ob):
    grid = (_NH // _TT,)
    return pl.pallas_call(
        _combine_body,
        grid=grid,
        in_specs=[
            pl.BlockSpec((_TT, _D), lambda i: (i, 0)),
            pl.BlockSpec((_TT, _D), lambda i: (i + _NH // _TT, 0)),
            pl.BlockSpec((_TT, _K), lambda i: (i, 0)),
            pl.BlockSpec((_D, _D), lambda i: (0, 0)),
            pl.BlockSpec((1, _D), lambda i: (0, 0)),
        ],
        out_specs=pl.BlockSpec((_TT, _D), lambda i: (i, 0)),
        out_shape=jax.ShapeDtypeStruct((_NH, _D), jnp.float32),
        compiler_params=pltpu.CompilerParams(
            dimension_semantics=("arbitrary",),
        ),
    )(rows, rows, gates, wobf, ob)


@jax.jit
def kernel(x, router_kernel, router_bias, expert_kernels, expert_biases,
           out_kernel, out_bias):
    b, s, d = x.shape
    xf = x.reshape(b * s, d)
    xbf = xf.astype(jnp.bfloat16)
    rkbf = router_kernel.astype(jnp.bfloat16)
    ekbf = expert_kernels.astype(jnp.bfloat16)
    wobf = out_kernel.astype(jnp.bfloat16)
    rb = router_bias.reshape(1, _E)
    ob = out_bias.reshape(1, _D)
    eb3 = expert_biases.reshape(_E, 1, _D)

    dest, gates, tmeta = _route(xbf, rkbf, rb)
    tmeta2 = tmeta.reshape(2, 64)

    outs = []
    for h in range(2):
        dest_h = dest[h * _NH:(h + 1) * _NH]
        idx_h = dest_h.T.reshape(_NW, _NCH, _CH)
        te_h = tmeta2[h, :_GH]
        gates_h = gates[h * _NH:(h + 1) * _NH]
        sc_scatter, sc_gather = _sc_kernels(h)
        xs = sc_scatter(xf, idx_h)
        ys = _expert_gemm(te_h, xs, ekbf, eb3)
        rows = sc_gather(ys, idx_h)
        outs.append(_combine(rows, gates_h, wobf, ob))
    return jnp.concatenate(outs, axis=0).reshape(b, s, d)


# speedup vs baseline: 1.1562x; 1.1562x over previous
import functools

import jax
import jax.numpy as jnp
from jax import lax
from jax.experimental import pallas as pl
from jax.experimental.pallas import tpu as pltpu
from jax.experimental.pallas import tpu_sc as plsc

_E = 8
_D = 1024
_N = 4096
_K = 2
_A = _N * _K
_TG = 512
_G = _A // _TG + _E
_S = _G * _TG
_CT = 512
_NCK = _N // _CT
_TT = 512

_NW = 32
_RPW = _A // _NW
_CH = 32
_NCH = _RPW // _CH


def _router_body(x_ref, rk_ref, rb_ref, dest_ref, gates_ref, tmeta_ref):
    x = x_ref[...]
    logits = jnp.dot(x, rk_ref[...], preferred_element_type=jnp.float32)
    logits = logits + rb_ref[...]
    m = jnp.max(logits, axis=-1, keepdims=True)
    ex = jnp.exp(logits - m)
    probs = ex / jnp.sum(ex, axis=-1, keepdims=True)

    lane = lax.broadcasted_iota(jnp.int32, probs.shape, 1)
    p1 = jnp.max(probs, axis=-1, keepdims=True)
    i1 = jnp.min(jnp.where(probs == p1, lane, _E), axis=-1, keepdims=True)
    probs2 = jnp.where(lane == i1, -jnp.inf, probs)
    p2 = jnp.max(probs2, axis=-1, keepdims=True)
    i2 = jnp.min(jnp.where(probs2 == p2, lane, _E), axis=-1, keepdims=True)
    denom = p1 + p2
    gates_ref[...] = jnp.concatenate([p1 / denom, p2 / denom], axis=1)

    oh0 = (lane == i1).astype(jnp.float32)
    oh1 = (lane == i2).astype(jnp.float32)

    r = lax.broadcasted_iota(jnp.int32, (_CT, _CT), 0)
    c = lax.broadcasted_iota(jnp.int32, (_CT, _CT), 1)
    tril = (c < r).astype(jnp.bfloat16)
    oh01 = jnp.concatenate([oh0, oh1], axis=1)
    chunks = [oh01[i * _CT:(i + 1) * _CT] for i in range(_NCK)]
    big = jnp.concatenate(chunks, axis=1).astype(jnp.bfloat16)
    scan = jnp.dot(tril, big, preferred_element_type=jnp.float32)
    carry = jnp.zeros((1, 2 * _E), jnp.float32)
    parts = []
    for i in range(_NCK):
        parts.append(scan[:, i * 2 * _E:(i + 1) * 2 * _E] + carry)
        carry = carry + jnp.sum(chunks[i], axis=0, keepdims=True)
    ranks01 = jnp.concatenate(parts, axis=0)
    cnt0 = carry[:, :_E]
    ranks0 = ranks01[:, :_E]
    ranks1 = ranks01[:, _E:] + cnt0
    counts = cnt0 + carry[:, _E:]

    ptiles = jnp.floor((counts + (_TG - 1.0)) * (1.0 / _TG))
    rr = lax.broadcasted_iota(jnp.int32, (_E, _E), 0)
    cc = lax.broadcasted_iota(jnp.int32, (_E, _E), 1)
    triu = (rr < cc).astype(jnp.bfloat16)
    tstart = jnp.dot(ptiles.astype(jnp.bfloat16), triu,
                     preferred_element_type=jnp.float32)
    toff = tstart * float(_TG)

    d0 = jnp.sum(oh0 * (toff + ranks0), axis=1, keepdims=True)
    d1 = jnp.sum(oh1 * (toff + ranks1), axis=1, keepdims=True)
    dest_ref[...] = jnp.concatenate([d0, d1], axis=1).astype(jnp.int32)

    gi = lax.broadcasted_iota(jnp.int32, (64, _E), 0).astype(jnp.float32)
    te = jnp.sum((gi >= tstart).astype(jnp.float32), axis=1, keepdims=True)
    tmeta_ref[...] = (te - 1.0).astype(jnp.int32)


def _route(xbf, rkbf, rb):
    return pl.pallas_call(
        _router_body,
        in_specs=[
            pl.BlockSpec((_N, _D), lambda: (0, 0)),
            pl.BlockSpec((_D, _E), lambda: (0, 0)),
            pl.BlockSpec((1, _E), lambda: (0, 0)),
        ],
        out_specs=[
            pl.BlockSpec((_N, _K), lambda: (0, 0)),
            pl.BlockSpec((_N, _K), lambda: (0, 0)),
            pl.BlockSpec((64, 1), lambda: (0, 0)),
        ],
        out_shape=[
            jax.ShapeDtypeStruct((_N, _K), jnp.int32),
            jax.ShapeDtypeStruct((_N, _K), jnp.float32),
            jax.ShapeDtypeStruct((64, 1), jnp.int32),
        ],
    )(xbf, rkbf, rb)


@functools.lru_cache(maxsize=None)
def _sc_kernels():
    mesh = plsc.VectorSubcoreMesh(core_axis_name="c", subcore_axis_name="s")
    cp = pltpu.CompilerParams(use_tc_tiling_on_sc=True)

    @functools.partial(
        pl.kernel, mesh=mesh, compiler_params=cp,
        out_type=jax.ShapeDtypeStruct((_S, _D), jnp.float32),
        scratch_types=[
            pltpu.VMEM((_NCH, _CH), jnp.int32),
            pltpu.VMEM((_CH, _D), jnp.float32),
            pltpu.VMEM((_CH, _D), jnp.float32),
            pltpu.SemaphoreType.DMA,
            pltpu.SemaphoreType.DMA,
            pltpu.SemaphoreType.DMA,
        ],
    )
    def sc_scatter(x_hbm, idx_hbm, xs_hbm, idx_v, buf_a, buf_b,
                   sem_a, sem_b, sem_w):
        wid = lax.axis_index("s") * 2 + lax.axis_index("c")
        base = wid * _RPW
        pltpu.sync_copy(idx_hbm.at[wid], idx_v)
        bufs, sems = (buf_a, buf_b), (sem_a, sem_b)

        def src(j):
            return (base + j * _CH) % _N

        rd = [None, None]
        rd[0] = pltpu.async_copy(x_hbm.at[pl.ds(src(0), _CH)], buf_a, sem_a)
        wr = None
        for j in range(_NCH):
            rd[j % 2].wait()
            if wr is not None:
                wr.wait()
            if j + 1 < _NCH:
                rd[(j + 1) % 2] = pltpu.async_copy(
                    x_hbm.at[pl.ds(src(j + 1), _CH)],
                    bufs[(j + 1) % 2], sems[(j + 1) % 2])
            wr = pltpu.async_copy(bufs[j % 2], xs_hbm.at[idx_v.at[j]], sem_w)
        wr.wait()

    @functools.partial(
        pl.kernel, mesh=mesh, compiler_params=cp,
        out_type=jax.ShapeDtypeStruct((_A, _D), jnp.float32),
        scratch_types=[
            pltpu.VMEM((_NCH, _CH), jnp.int32),
            pltpu.VMEM((_CH, _D), jnp.float32),
            pltpu.VMEM((_CH, _D), jnp.float32),
            pltpu.SemaphoreType.DMA,
            pltpu.SemaphoreType.DMA,
            pltpu.SemaphoreType.DMA,
        ],
    )
    def sc_gather(ys_hbm, idx_hbm, rows_hbm, idx_v, buf_a, buf_b,
                  sem_a, sem_b, sem_w):
        wid = lax.axis_index("s") * 2 + lax.axis_index("c")
        base = wid * _RPW
        pltpu.sync_copy(idx_hbm.at[wid], idx_v)
        bufs, sems = (buf_a, buf_b), (sem_a, sem_b)
        rd = [None, None]
        rd[0] = pltpu.async_copy(ys_hbm.at[idx_v.at[0]], buf_a, sem_a)
        wr = None
        for j in range(_NCH):
            rd[j % 2].wait()
            if wr is not None:
                wr.wait()
            if j + 1 < _NCH:
                rd[(j + 1) % 2] = pltpu.async_copy(
                    ys_hbm.at[idx_v.at[j + 1]],
                    bufs[(j + 1) % 2], sems[(j + 1) % 2])
            wr = pltpu.async_copy(bufs[j % 2],
                                  rows_hbm.at[pl.ds(base + j * _CH, _CH)],
                                  sem_w)
        wr.wait()

    return sc_scatter, sc_gather


def _expert_body(te_ref, xs_ref, ek_ref, eb_ref, ys_ref):
    h = jnp.dot(xs_ref[...].astype(jnp.bfloat16), ek_ref[0],
                preferred_element_type=jnp.float32)
    h = h + eb_ref[0]
    ys_ref[...] = jax.nn.gelu(h)


def _expert_gemm(tile_expert, xs, ekbf, eb):
    grid_spec = pltpu.PrefetchScalarGridSpec(
        num_scalar_prefetch=1,
        grid=(_G,),
        in_specs=[
            pl.BlockSpec((_TG, _D), lambda g, te: (g, 0)),
            pl.BlockSpec((1, _D, _D), lambda g, te: (te[g], 0, 0)),
            pl.BlockSpec((1, 1, _D), lambda g, te: (te[g], 0, 0)),
        ],
        out_specs=pl.BlockSpec((_TG, _D), lambda g, te: (g, 0)),
    )
    return pl.pallas_call(
        _expert_body,
        grid_spec=grid_spec,
        out_shape=jax.ShapeDtypeStruct((_S, _D), jnp.float32),
        compiler_params=pltpu.CompilerParams(
            dimension_semantics=("arbitrary",),
        ),
    )(tile_expert, xs, ekbf, eb)


def _combine_body(r0_ref, r1_ref, gates_ref, wo_ref, ob_ref, out_ref):
    g = gates_ref[...]
    comb = r0_ref[...] * g[:, 0:1] + r1_ref[...] * g[:, 1:2]
    out = jnp.dot(comb.astype(jnp.bfloat16), wo_ref[...],
                  preferred_element_type=jnp.float32)
    out_ref[...] = out + ob_ref[...]


def _combine(rows, gates, wobf, ob):
    grid = (_N // _TT,)
    return pl.pallas_call(
        _combine_body,
        grid=grid,
        in_specs=[
            pl.BlockSpec((_TT, _D), lambda i: (i, 0)),
            pl.BlockSpec((_TT, _D), lambda i: (i + _N // _TT, 0)),
            pl.BlockSpec((_TT, _K), lambda i: (i, 0)),
            pl.BlockSpec((_D, _D), lambda i: (0, 0)),
            pl.BlockSpec((1, _D), lambda i: (0, 0)),
        ],
        out_specs=pl.BlockSpec((_TT, _D), lambda i: (i, 0)),
        out_shape=jax.ShapeDtypeStruct((_N, _D), jnp.float32),
        compiler_params=pltpu.CompilerParams(
            dimension_semantics=("arbitrary",),
        ),
    )(rows, rows, gates, wobf, ob)


@jax.jit
def kernel(x, router_kernel, router_bias, expert_kernels, expert_biases,
           out_kernel, out_bias):
    b, s, d = x.shape
    xf = x.reshape(b * s, d)
    xbf = xf.astype(jnp.bfloat16)
    rkbf = router_kernel.astype(jnp.bfloat16)
    ekbf = expert_kernels.astype(jnp.bfloat16)
    wobf = out_kernel.astype(jnp.bfloat16)
    rb = router_bias.reshape(1, _E)
    ob = out_bias.reshape(1, _D)
    eb3 = expert_biases.reshape(_E, 1, _D)

    dest, gates, tmeta = _route(xbf, rkbf, rb)
    idx = dest.T.reshape(_NW, _NCH, _CH)
    tile_expert = tmeta.reshape(64)[:_G]

    sc_scatter, sc_gather = _sc_kernels()
    xs = sc_scatter(xf, idx)
    ys = _expert_gemm(tile_expert, xs, ekbf, eb3)
    rows = sc_gather(ys, idx)
    out = _combine(rows, gates, wobf, ob)
    return out.reshape(b, s, d)


# speedup vs baseline: 1.1583x; 1.0018x over previous
import functools

import jax
import jax.numpy as jnp
from jax import lax
from jax.experimental import pallas as pl
from jax.experimental.pallas import tpu as pltpu
from jax.experimental.pallas import tpu_sc as plsc

_E = 8
_D = 1024
_N = 4096
_K = 2
_A = _N * _K
_TG = 512
_G = _A // _TG + _E
_S = _G * _TG
_CT = 512
_NCK = _N // _CT
_TT = 512

_NW = 32
_RPW = _A // _NW
_CH = 32
_NCH = _RPW // _CH


def _router_body(x_ref, rk_ref, rb_ref, dest_ref, gates_ref, tmeta_ref):
    x = x_ref[...]
    logits = jnp.dot(x, rk_ref[...], preferred_element_type=jnp.float32)
    logits = logits + rb_ref[...]
    m = jnp.max(logits, axis=-1, keepdims=True)
    ex = jnp.exp(logits - m)
    probs = ex / jnp.sum(ex, axis=-1, keepdims=True)

    lane = lax.broadcasted_iota(jnp.int32, probs.shape, 1)
    p1 = jnp.max(probs, axis=-1, keepdims=True)
    i1 = jnp.min(jnp.where(probs == p1, lane, _E), axis=-1, keepdims=True)
    probs2 = jnp.where(lane == i1, -jnp.inf, probs)
    p2 = jnp.max(probs2, axis=-1, keepdims=True)
    i2 = jnp.min(jnp.where(probs2 == p2, lane, _E), axis=-1, keepdims=True)
    denom = p1 + p2
    gates_ref[...] = jnp.concatenate([p1 / denom, p2 / denom], axis=1)

    oh0 = (lane == i1).astype(jnp.float32)
    oh1 = (lane == i2).astype(jnp.float32)

    r = lax.broadcasted_iota(jnp.int32, (_CT, _CT), 0)
    c = lax.broadcasted_iota(jnp.int32, (_CT, _CT), 1)
    tril = (c < r).astype(jnp.bfloat16)
    oh01 = jnp.concatenate([oh0, oh1], axis=1)
    chunks = [oh01[i * _CT:(i + 1) * _CT] for i in range(_NCK)]
    big = jnp.concatenate(chunks, axis=1).astype(jnp.bfloat16)
    scan = jnp.dot(tril, big, preferred_element_type=jnp.float32)
    carry = jnp.zeros((1, 2 * _E), jnp.float32)
    parts = []
    for i in range(_NCK):
        parts.append(scan[:, i * 2 * _E:(i + 1) * 2 * _E] + carry)
        carry = carry + jnp.sum(chunks[i], axis=0, keepdims=True)
    ranks01 = jnp.concatenate(parts, axis=0)
    cnt0 = carry[:, :_E]
    ranks0 = ranks01[:, :_E]
    ranks1 = ranks01[:, _E:] + cnt0
    counts = cnt0 + carry[:, _E:]

    ptiles = jnp.floor((counts + (_TG - 1.0)) * (1.0 / _TG))
    rr = lax.broadcasted_iota(jnp.int32, (_E, _E), 0)
    cc = lax.broadcasted_iota(jnp.int32, (_E, _E), 1)
    triu = (rr < cc).astype(jnp.bfloat16)
    tstart = jnp.dot(ptiles.astype(jnp.bfloat16), triu,
                     preferred_element_type=jnp.float32)
    toff = tstart * float(_TG)

    d0 = jnp.sum(oh0 * (toff + ranks0), axis=1, keepdims=True)
    d1 = jnp.sum(oh1 * (toff + ranks1), axis=1, keepdims=True)
    dest_ref[...] = jnp.concatenate([d0, d1], axis=1).astype(jnp.int32)

    gi = lax.broadcasted_iota(jnp.int32, (64, _E), 0).astype(jnp.float32)
    te = jnp.sum((gi >= tstart).astype(jnp.float32), axis=1, keepdims=True)
    tmeta_ref[...] = (te - 1.0).astype(jnp.int32)


def _route(xbf, rkbf, rb):
    return pl.pallas_call(
        _router_body,
        in_specs=[
            pl.BlockSpec((_N, _D), lambda: (0, 0)),
            pl.BlockSpec((_D, _E), lambda: (0, 0)),
            pl.BlockSpec((1, _E), lambda: (0, 0)),
        ],
        out_specs=[
            pl.BlockSpec((_N, _K), lambda: (0, 0)),
            pl.BlockSpec((_N, _K), lambda: (0, 0)),
            pl.BlockSpec((64, 1), lambda: (0, 0)),
        ],
        out_shape=[
            jax.ShapeDtypeStruct((_N, _K), jnp.int32),
            jax.ShapeDtypeStruct((_N, _K), jnp.float32),
            jax.ShapeDtypeStruct((64, 1), jnp.int32),
        ],
    )(xbf, rkbf, rb)


@functools.lru_cache(maxsize=None)
def _sc_kernels():
    mesh = plsc.VectorSubcoreMesh(core_axis_name="c", subcore_axis_name="s")
    cp = pltpu.CompilerParams(use_tc_tiling_on_sc=True)

    @functools.partial(
        pl.kernel, mesh=mesh, compiler_params=cp,
        out_type=jax.ShapeDtypeStruct((_S, _D), jnp.float32),
        scratch_types=[
            pltpu.VMEM((_NCH, _CH), jnp.int32),
            pltpu.VMEM((_CH, _D), jnp.float32),
            pltpu.VMEM((_CH, _D), jnp.float32),
            pltpu.SemaphoreType.DMA,
            pltpu.SemaphoreType.DMA,
            pltpu.SemaphoreType.DMA,
            pltpu.SemaphoreType.DMA,
        ],
    )
    def sc_scatter(x_hbm, idx_hbm, xs_hbm, idx_v, buf_a, buf_b,
                   sem_a, sem_b, sem_w0, sem_w1):
        wid = lax.axis_index("s") * 2 + lax.axis_index("c")
        base = wid * _RPW
        pltpu.sync_copy(idx_hbm.at[wid], idx_v)
        bufs, sems = (buf_a, buf_b), (sem_a, sem_b)
        wsems = (sem_w0, sem_w1)

        def src(j):
            return (base + j * _CH) % _N

        rd = [None, None]
        wr = [None, None]
        rd[0] = pltpu.async_copy(x_hbm.at[pl.ds(src(0), _CH)], buf_a, sem_a)
        for j in range(_NCH):
            rd[j % 2].wait()
            if wr[j % 2] is not None:
                wr[j % 2].wait()
            if j + 1 < _NCH:
                if wr[(j + 1) % 2] is not None:
                    wr[(j + 1) % 2].wait()
                    wr[(j + 1) % 2] = None
                rd[(j + 1) % 2] = pltpu.async_copy(
                    x_hbm.at[pl.ds(src(j + 1), _CH)],
                    bufs[(j + 1) % 2], sems[(j + 1) % 2])
            wr[j % 2] = pltpu.async_copy(bufs[j % 2], xs_hbm.at[idx_v.at[j]],
                                         wsems[j % 2])
        for w in wr:
            if w is not None:
                w.wait()

    @functools.partial(
        pl.kernel, mesh=mesh, compiler_params=cp,
        out_type=jax.ShapeDtypeStruct((_A, _D), jnp.float32),
        scratch_types=[
            pltpu.VMEM((_NCH, _CH), jnp.int32),
            pltpu.VMEM((_CH, _D), jnp.float32),
            pltpu.VMEM((_CH, _D), jnp.float32),
            pltpu.SemaphoreType.DMA,
            pltpu.SemaphoreType.DMA,
            pltpu.SemaphoreType.DMA,
            pltpu.SemaphoreType.DMA,
        ],
    )
    def sc_gather(ys_hbm, idx_hbm, rows_hbm, idx_v, buf_a, buf_b,
                  sem_a, sem_b, sem_w0, sem_w1):
        wid = lax.axis_index("s") * 2 + lax.axis_index("c")
        base = wid * _RPW
        pltpu.sync_copy(idx_hbm.at[wid], idx_v)
        bufs, sems = (buf_a, buf_b), (sem_a, sem_b)
        wsems = (sem_w0, sem_w1)
        rd = [None, None]
        wr = [None, None]
        rd[0] = pltpu.async_copy(ys_hbm.at[idx_v.at[0]], buf_a, sem_a)
        for j in range(_NCH):
            rd[j % 2].wait()
            if wr[j % 2] is not None:
                wr[j % 2].wait()
            if j + 1 < _NCH:
                if wr[(j + 1) % 2] is not None:
                    wr[(j + 1) % 2].wait()
                    wr[(j + 1) % 2] = None
                rd[(j + 1) % 2] = pltpu.async_copy(
                    ys_hbm.at[idx_v.at[j + 1]],
                    bufs[(j + 1) % 2], sems[(j + 1) % 2])
            wr[j % 2] = pltpu.async_copy(bufs[j % 2],
                                         rows_hbm.at[pl.ds(base + j * _CH, _CH)],
                                         wsems[j % 2])
        for w in wr:
            if w is not None:
                w.wait()

    return sc_scatter, sc_gather


def _expert_body(te_ref, xs_ref, ek_ref, eb_ref, ys_ref):
    h = jnp.dot(xs_ref[...].astype(jnp.bfloat16), ek_ref[0],
                preferred_element_type=jnp.float32)
    h = h + eb_ref[0]
    ys_ref[...] = jax.nn.gelu(h)


def _expert_gemm(tile_expert, xs, ekbf, eb):
    grid_spec = pltpu.PrefetchScalarGridSpec(
        num_scalar_prefetch=1,
        grid=(_G,),
        in_specs=[
            pl.BlockSpec((_TG, _D), lambda g, te: (g, 0)),
            pl.BlockSpec((1, _D, _D), lambda g, te: (te[g], 0, 0)),
            pl.BlockSpec((1, 1, _D), lambda g, te: (te[g], 0, 0)),
        ],
        out_specs=pl.BlockSpec((_TG, _D), lambda g, te: (g, 0)),
    )
    return pl.pallas_call(
        _expert_body,
        grid_spec=grid_spec,
        out_shape=jax.ShapeDtypeStruct((_S, _D), jnp.float32),
        compiler_params=pltpu.CompilerParams(
            dimension_semantics=("arbitrary",),
        ),
    )(tile_expert, xs, ekbf, eb)


def _combine_body(r0_ref, r1_ref, gates_ref, wo_ref, ob_ref, out_ref):
    g = gates_ref[...]
    comb = r0_ref[...] * g[:, 0:1] + r1_ref[...] * g[:, 1:2]
    out = jnp.dot(comb.astype(jnp.bfloat16), wo_ref[...],
                  preferred_element_type=jnp.float32)
    out_ref[...] = out + ob_ref[...]


def _combine(rows, gates, wobf, ob):
    grid = (_N // _TT,)
    return pl.pallas_call(
        _combine_body,
        grid=grid,
        in_specs=[
            pl.BlockSpec((_TT, _D), lambda i: (i, 0)),
            pl.BlockSpec((_TT, _D), lambda i: (i + _N // _TT, 0)),
            pl.BlockSpec((_TT, _K), lambda i: (i, 0)),
            pl.BlockSpec((_D, _D), lambda i: (0, 0)),
            pl.BlockSpec((1, _D), lambda i: (0, 0)),
        ],
        out_specs=pl.BlockSpec((_TT, _D), lambda i: (i, 0)),
        out_shape=jax.ShapeDtypeStruct((_N, _D), jnp.float32),
        compiler_params=pltpu.CompilerParams(
            dimension_semantics=("arbitrary",),
        ),
    )(rows, rows, gates, wobf, ob)


@jax.jit
def kernel(x, router_kernel, router_bias, expert_kernels, expert_biases,
           out_kernel, out_bias):
    b, s, d = x.shape
    xf = x.reshape(b * s, d)
    xbf = xf.astype(jnp.bfloat16)
    rkbf = router_kernel.astype(jnp.bfloat16)
    ekbf = expert_kernels.astype(jnp.bfloat16)
    wobf = out_kernel.astype(jnp.bfloat16)
    rb = router_bias.reshape(1, _E)
    ob = out_bias.reshape(1, _D)
    eb3 = expert_biases.reshape(_E, 1, _D)

    dest, gates, tmeta = _route(xbf, rkbf, rb)
    idx = dest.T.reshape(_NW, _NCH, _CH)
    tile_expert = tmeta.reshape(64)[:_G]

    sc_scatter, sc_gather = _sc_kernels()
    xs = sc_scatter(xf, idx)
    ys = _expert_gemm(tile_expert, xs, ekbf, eb3)
    rows = sc_gather(ys, idx)
    out = _combine(rows, gates, wobf, ob)
    return out.reshape(b, s, d)
